# Initial kernel scaffold; baseline (speedup 1.0000x reference)
import jax
import jax.numpy as jnp
from jax.experimental import pallas as pl


def kernel(x_paper, x_author, edge_index_writes, edge_index_cites, edge_attr_cites, delta_t_cites, Wp, bp, Wa, ba, Wl, bl, Wr, Wq, bq, Wk, bk, Wv, bv, We, be, lam, Wi, bi):
    raise NotImplementedError("write your pallas kernel here")


# speedup vs baseline: 1.2429x; 1.2429x over previous
import functools
import math

import jax
import jax.numpy as jnp
from jax.experimental import pallas as pl
from jax.experimental.pallas import tpu as pltpu

N_NODES = 50000
HD = 64
ED = 16
E_CITES = 800000
E_WRITES = 800000

_NODE_BLK = 2000
_EDGE_BLK = 6400


def _nodes_body(xpa_ref, xau_ref, Wp_ref, bp_ref, Wa_ref, ba_ref,
                Wq_ref, bq_ref, Wk_ref, bk_ref, Wv_ref, bv_ref, Wr_ref,
                xa_out, q_out, k_out, v_out, xpr_out):
    xp = jnp.maximum(xpa_ref[...] @ Wp_ref[...] + bp_ref[...][None, :], 0.0)
    xa = jnp.maximum(xau_ref[...] @ Wa_ref[...] + ba_ref[...][None, :], 0.0)
    xa_out[...] = xa
    q_out[...] = xp @ Wq_ref[...] + bq_ref[...][None, :]
    k_out[...] = xp @ Wk_ref[...] + bk_ref[...][None, :]
    v_out[...] = xp @ Wv_ref[...] + bv_ref[...][None, :]
    xpr_out[...] = xp @ Wr_ref[...]


def _nodes_stage(x_paper, x_author, Wp, bp, Wa, ba, Wq, bq, Wk, bk, Wv, bv, Wr):
    n = x_paper.shape[0]
    grid = (n // _NODE_BLK,)
    row_spec = pl.BlockSpec((_NODE_BLK, 128), lambda i: (i, 0))
    w128 = pl.BlockSpec((128, HD), lambda i: (0, 0))
    w64 = pl.BlockSpec((HD, HD), lambda i: (0, 0))
    b_spec = pl.BlockSpec((HD,), lambda i: (0,))
    out_spec = pl.BlockSpec((_NODE_BLK, HD), lambda i: (i, 0))
    out_shape = jax.ShapeDtypeStruct((n, HD), jnp.float32)
    return pl.pallas_call(
        _nodes_body,
        grid=grid,
        in_specs=[row_spec, row_spec, w128, b_spec, w128, b_spec,
                  w64, b_spec, w64, b_spec, w64, b_spec, w64],
        out_specs=[out_spec] * 5,
        out_shape=[out_shape] * 5,
    )(x_paper, x_author, Wp, bp, Wa, ba, Wq, bq, Wk, bk, Wv, bv, Wr)


def _edge_body(qk_ref, vsrc_ref, ea_ref, dt_ref, We_ref, be_ref,
               Wie_ref, wit_ref, bi4_ref, nl_ref,
               s_out, sv_out, le_out):
    ea = ea_ref[...]
    t = ea @ We_ref[...] + be_ref[...][None, :]
    a = jnp.sum(qk_ref[...] * t, axis=1, keepdims=True) * (1.0 / math.sqrt(HD))
    dt = dt_ref[...]
    dec = jnp.exp(nl_ref[0, 0] * dt)
    s = jnp.exp(a * dec)
    s_out[...] = s
    sv_out[...] = s * vsrc_ref[...]
    le_out[...] = ea @ Wie_ref[...] + dt * wit_ref[...] + bi4_ref[...]


def _edge_stage(qk, vsrc, ea, dt, We, be, Wie4, wit4, bi4, neg_lam):
    e = qk.shape[0]
    grid = (e // _EDGE_BLK,)
    rows64 = pl.BlockSpec((_EDGE_BLK, HD), lambda i: (i, 0))
    rows16 = pl.BlockSpec((_EDGE_BLK, ED), lambda i: (i, 0))
    vec = pl.BlockSpec((_EDGE_BLK, 1), lambda i: (i, 0))
    const = lambda shape: pl.BlockSpec(shape, lambda i: (0,) * len(shape))
    return pl.pallas_call(
        _edge_body,
        grid=grid,
        in_specs=[rows64, rows64, rows16, vec, const((ED, HD)), const((HD,)),
                  const((ED, 4)), const((1, 4)), const((1, 4)), const((1, 1))],
        out_specs=[vec, rows64, pl.BlockSpec((_EDGE_BLK, 4), lambda i: (i, 0))],
        out_shape=[jax.ShapeDtypeStruct((e, 1), jnp.float32),
                   jax.ShapeDtypeStruct((e, HD), jnp.float32),
                   jax.ShapeDtypeStruct((e, 4), jnp.float32)],
    )(qk, vsrc, ea, dt, We, be, Wie4, wit4, bi4, neg_lam)


def _hnode_body(msum_ref, cnt_ref, xpr_ref, svsum_ref, denom_ref,
                Wl_ref, bl_ref, Wpack_ref, e6_ref, hsd_out):
    mean = msum_ref[...] / jnp.maximum(cnt_ref[...], 1.0)
    sage = mean @ Wl_ref[...] + bl_ref[...][None, :] + xpr_ref[...]
    rden = 1.0 / jnp.maximum(denom_ref[...], 1e-30)
    h = jnp.maximum(sage + svsum_ref[...] * rden, 0.0)
    hsd_out[...] = h @ Wpack_ref[...] + rden * e6_ref[...]


def _hnode_stage(msum, cnt, xpr, svsum, denom, Wl, bl, Wpack, e6):
    n = msum.shape[0]
    grid = (n // _NODE_BLK,)
    rows64 = pl.BlockSpec((_NODE_BLK, HD), lambda i: (i, 0))
    vec = pl.BlockSpec((_NODE_BLK, 1), lambda i: (i, 0))
    const = lambda shape: pl.BlockSpec(shape, lambda i: (0,) * len(shape))
    return pl.pallas_call(
        _hnode_body,
        grid=grid,
        in_specs=[rows64, vec, rows64, rows64, vec,
                  const((HD, HD)), const((HD,)), const((HD, 16)), const((1, 16))],
        out_specs=pl.BlockSpec((_NODE_BLK, 16), lambda i: (i, 0)),
        out_shape=jax.ShapeDtypeStruct((n, 16), jnp.float32),
    )(msum, cnt, xpr, svsum, denom, Wl, bl, Wpack, e6)


def kernel(x_paper, x_author, edge_index_writes, edge_index_cites,
           edge_attr_cites, delta_t_cites, Wp, bp, Wa, ba, Wl, bl, Wr,
           Wq, bq, Wk, bk, Wv, bv, We, be, lam, Wi, bi):
    src_w = edge_index_writes[0]
    dst_w = edge_index_writes[1]
    src_c = edge_index_cites[0]
    dst_c = edge_index_cites[1]
    dt = delta_t_cites

    xa, q, k, v, xpr = _nodes_stage(
        x_paper, x_author, Wp, bp, Wa, ba, Wq, bq, Wk, bk, Wv, bv, Wr)

    msum = jax.ops.segment_sum(jnp.take(xa, src_w, axis=0), dst_w,
                               num_segments=N_NODES)
    cnt = jax.ops.segment_sum(jnp.ones((E_WRITES,), jnp.float32), dst_w,
                              num_segments=N_NODES)[:, None]

    qk = jnp.take(q, dst_c, axis=0) * jnp.take(k, src_c, axis=0)
    vsrc = jnp.take(v, src_c, axis=0)

    Wie4 = jnp.zeros((ED, 4), jnp.float32).at[:, :3].set(Wi[2 * HD:2 * HD + ED, :])
    wit4 = jnp.zeros((1, 4), jnp.float32).at[0, :3].set(Wi[2 * HD + ED, :])
    bi4 = jnp.zeros((1, 4), jnp.float32).at[0, :3].set(bi)
    neg_lam = (-jnp.abs(lam)).reshape(1, 1)
    s, sv, le = _edge_stage(qk, vsrc, edge_attr_cites, dt, We, be,
                            Wie4, wit4, bi4, neg_lam)

    denom = jax.ops.segment_sum(s[:, 0], dst_c, num_segments=N_NODES)[:, None]
    svsum = jax.ops.segment_sum(sv, dst_c, num_segments=N_NODES)

    Wpack = (jnp.zeros((HD, 16), jnp.float32)
             .at[:, 0:3].set(Wi[:HD, :])
             .at[:, 3:6].set(Wi[HD:2 * HD, :]))
    e6 = jnp.zeros((1, 16), jnp.float32).at[0, 6].set(1.0)
    hsd = _hnode_stage(msum, cnt, xpr, svsum, denom, Wl, bl, Wpack, e6)

    logits = (jnp.take(hsd[:, 0:3], src_c, axis=0)
              + jnp.take(hsd[:, 3:6], dst_c, axis=0) + le[:, :3])
    alpha_n = s[:, 0] * jnp.take(hsd[:, 6], dst_c, axis=0)
    return (logits, jax.lax.stop_gradient(alpha_n))


# speedup vs baseline: 1.4452x; 1.1628x over previous
import functools
import math

import jax
import jax.numpy as jnp
from jax import lax
from jax.experimental import pallas as pl
from jax.experimental.pallas import tpu as pltpu
from jax.experimental.pallas import tpu_sc as plsc

N_NODES = 50000
HD = 64
ED = 16
E_CITES = 800000
E_WRITES = 800000

_NODE_BLK = 2000
_EDGE_BLK = 6400

_NW = 32
_SC_CHUNK = 200


def _sc_mesh():
    return plsc.VectorSubcoreMesh(core_axis_name="c", subcore_axis_name="s")


def _qkv_gather_body(q_hbm, k_hbm, v_hbm, dst_hbm, src_hbm,
                     qk_hbm, vsrc_hbm,
                     dstv, srcv, qr, kr, vr, sem0, sem1, sem2):
    wid = lax.axis_index("s") * 2 + lax.axis_index("c")
    per_w = E_CITES // _NW
    nchunk = per_w // _SC_CHUNK
    C = _SC_CHUNK

    def chunk(ci, carry):
        base = wid * per_w + ci * C
        pltpu.sync_copy(dst_hbm.at[pl.ds(base, C)], dstv)
        pltpu.sync_copy(src_hbm.at[pl.ds(base, C)], srcv)
        cp0 = pltpu.async_copy(q_hbm.at[dstv], qr, sem0)
        cp1 = pltpu.async_copy(k_hbm.at[srcv], kr, sem1)
        cp2 = pltpu.async_copy(v_hbm.at[srcv], vr, sem2)
        cp0.wait()
        cp1.wait()

        def mul(r, carry2):
            for g in range(4):
                sl = pl.ds(g * 16, 16)
                qr[r, sl] = qr[r, sl] * kr[r, sl]
            return carry2

        lax.fori_loop(0, C, mul, 0)
        pltpu.sync_copy(qr, qk_hbm.at[pl.ds(base, C)])
        cp2.wait()
        pltpu.sync_copy(vr, vsrc_hbm.at[pl.ds(base, C)])
        return carry

    lax.fori_loop(0, nchunk, chunk, 0)


def _qkv_gather(q, k, v, dst_c, src_c):
    C = _SC_CHUNK
    f = pl.kernel(
        _qkv_gather_body,
        mesh=_sc_mesh(),
        compiler_params=pltpu.CompilerParams(use_tc_tiling_on_sc=False),
        out_type=[jax.ShapeDtypeStruct((E_CITES, HD), jnp.float32),
                  jax.ShapeDtypeStruct((E_CITES, HD), jnp.float32)],
        scratch_types=[
            pltpu.VMEM((C,), jnp.int32),
            pltpu.VMEM((C,), jnp.int32),
            pltpu.VMEM((C, HD), jnp.float32),
            pltpu.VMEM((C, HD), jnp.float32),
            pltpu.VMEM((C, HD), jnp.float32),
            pltpu.SemaphoreType.DMA,
            pltpu.SemaphoreType.DMA,
            pltpu.SemaphoreType.DMA,
        ],
    )
    return f(q, k, v, dst_c, src_c)


def _nodes_body(xpa_ref, xau_ref, Wp_ref, bp_ref, Wa_ref, ba_ref,
                Wq_ref, bq_ref, Wk_ref, bk_ref, Wv_ref, bv_ref, Wr_ref,
                xa_out, q_out, k_out, v_out, xpr_out):
    xp = jnp.maximum(xpa_ref[...] @ Wp_ref[...] + bp_ref[...][None, :], 0.0)
    xa = jnp.maximum(xau_ref[...] @ Wa_ref[...] + ba_ref[...][None, :], 0.0)
    xa_out[...] = xa
    q_out[...] = xp @ Wq_ref[...] + bq_ref[...][None, :]
    k_out[...] = xp @ Wk_ref[...] + bk_ref[...][None, :]
    v_out[...] = xp @ Wv_ref[...] + bv_ref[...][None, :]
    xpr_out[...] = xp @ Wr_ref[...]


def _nodes_stage(x_paper, x_author, Wp, bp, Wa, ba, Wq, bq, Wk, bk, Wv, bv, Wr):
    n = x_paper.shape[0]
    grid = (n // _NODE_BLK,)
    row_spec = pl.BlockSpec((_NODE_BLK, 128), lambda i: (i, 0))
    w128 = pl.BlockSpec((128, HD), lambda i: (0, 0))
    w64 = pl.BlockSpec((HD, HD), lambda i: (0, 0))
    b_spec = pl.BlockSpec((HD,), lambda i: (0,))
    out_spec = pl.BlockSpec((_NODE_BLK, HD), lambda i: (i, 0))
    out_shape = jax.ShapeDtypeStruct((n, HD), jnp.float32)
    return pl.pallas_call(
        _nodes_body,
        grid=grid,
        in_specs=[row_spec, row_spec, w128, b_spec, w128, b_spec,
                  w64, b_spec, w64, b_spec, w64, b_spec, w64],
        out_specs=[out_spec] * 5,
        out_shape=[out_shape] * 5,
    )(x_paper, x_author, Wp, bp, Wa, ba, Wq, bq, Wk, bk, Wv, bv, Wr)


def _edge_body(qk_ref, vsrc_ref, ea_ref, dt_ref, We_ref, be_ref,
               Wie_ref, wit_ref, bi4_ref, nl_ref,
               s_out, sv_out, le_out):
    ea = ea_ref[...]
    t = ea @ We_ref[...] + be_ref[...][None, :]
    a = jnp.sum(qk_ref[...] * t, axis=1, keepdims=True) * (1.0 / math.sqrt(HD))
    dt = dt_ref[...]
    dec = jnp.exp(nl_ref[0, 0] * dt)
    s = jnp.exp(a * dec)
    s_out[...] = s
    sv_out[...] = s * vsrc_ref[...]
    le_out[...] = ea @ Wie_ref[...] + dt * wit_ref[...] + bi4_ref[...]


def _edge_stage(qk, vsrc, ea, dt, We, be, Wie4, wit4, bi4, neg_lam):
    e = qk.shape[0]
    grid = (e // _EDGE_BLK,)
    rows64 = pl.BlockSpec((_EDGE_BLK, HD), lambda i: (i, 0))
    rows16 = pl.BlockSpec((_EDGE_BLK, ED), lambda i: (i, 0))
    vec = pl.BlockSpec((_EDGE_BLK, 1), lambda i: (i, 0))
    const = lambda shape: pl.BlockSpec(shape, lambda i: (0,) * len(shape))
    return pl.pallas_call(
        _edge_body,
        grid=grid,
        in_specs=[rows64, rows64, rows16, vec, const((ED, HD)), const((HD,)),
                  const((ED, 4)), const((1, 4)), const((1, 4)), const((1, 1))],
        out_specs=[vec, rows64, pl.BlockSpec((_EDGE_BLK, 4), lambda i: (i, 0))],
        out_shape=[jax.ShapeDtypeStruct((e, 1), jnp.float32),
                   jax.ShapeDtypeStruct((e, HD), jnp.float32),
                   jax.ShapeDtypeStruct((e, 4), jnp.float32)],
    )(qk, vsrc, ea, dt, We, be, Wie4, wit4, bi4, neg_lam)


def _hnode_body(msum_ref, cnt_ref, xpr_ref, svsum_ref, denom_ref,
                Wl_ref, bl_ref, Wpack_ref, e6_ref, hsd_out):
    mean = msum_ref[...] / jnp.maximum(cnt_ref[...], 1.0)
    sage = mean @ Wl_ref[...] + bl_ref[...][None, :] + xpr_ref[...]
    rden = 1.0 / jnp.maximum(denom_ref[...], 1e-30)
    h = jnp.maximum(sage + svsum_ref[...] * rden, 0.0)
    hsd_out[...] = h @ Wpack_ref[...] + rden * e6_ref[...]


def _hnode_stage(msum, cnt, xpr, svsum, denom, Wl, bl, Wpack, e6):
    n = msum.shape[0]
    grid = (n // _NODE_BLK,)
    rows64 = pl.BlockSpec((_NODE_BLK, HD), lambda i: (i, 0))
    vec = pl.BlockSpec((_NODE_BLK, 1), lambda i: (i, 0))
    const = lambda shape: pl.BlockSpec(shape, lambda i: (0,) * len(shape))
    return pl.pallas_call(
        _hnode_body,
        grid=grid,
        in_specs=[rows64, vec, rows64, rows64, vec,
                  const((HD, HD)), const((HD,)), const((HD, 16)), const((1, 16))],
        out_specs=pl.BlockSpec((_NODE_BLK, 16), lambda i: (i, 0)),
        out_shape=jax.ShapeDtypeStruct((n, 16), jnp.float32),
    )(msum, cnt, xpr, svsum, denom, Wl, bl, Wpack, e6)


def kernel(x_paper, x_author, edge_index_writes, edge_index_cites,
           edge_attr_cites, delta_t_cites, Wp, bp, Wa, ba, Wl, bl, Wr,
           Wq, bq, Wk, bk, Wv, bv, We, be, lam, Wi, bi):
    src_w = edge_index_writes[0]
    dst_w = edge_index_writes[1]
    src_c = edge_index_cites[0]
    dst_c = edge_index_cites[1]
    dt = delta_t_cites

    xa, q, k, v, xpr = _nodes_stage(
        x_paper, x_author, Wp, bp, Wa, ba, Wq, bq, Wk, bk, Wv, bv, Wr)

    msum = jax.ops.segment_sum(jnp.take(xa, src_w, axis=0), dst_w,
                               num_segments=N_NODES)
    cnt = jax.ops.segment_sum(jnp.ones((E_WRITES,), jnp.float32), dst_w,
                              num_segments=N_NODES)[:, None]

    qk, vsrc = _qkv_gather(q, k, v, dst_c, src_c)

    Wie4 = jnp.zeros((ED, 4), jnp.float32).at[:, :3].set(Wi[2 * HD:2 * HD + ED, :])
    wit4 = jnp.zeros((1, 4), jnp.float32).at[0, :3].set(Wi[2 * HD + ED, :])
    bi4 = jnp.zeros((1, 4), jnp.float32).at[0, :3].set(bi)
    neg_lam = (-jnp.abs(lam)).reshape(1, 1)
    s, sv, le = _edge_stage(qk, vsrc, edge_attr_cites, dt, We, be,
                            Wie4, wit4, bi4, neg_lam)

    denom = jax.ops.segment_sum(s[:, 0], dst_c, num_segments=N_NODES)[:, None]
    svsum = jax.ops.segment_sum(sv, dst_c, num_segments=N_NODES)

    Wpack = (jnp.zeros((HD, 16), jnp.float32)
             .at[:, 0:3].set(Wi[:HD, :])
             .at[:, 3:6].set(Wi[HD:2 * HD, :]))
    e6 = jnp.zeros((1, 16), jnp.float32).at[0, 6].set(1.0)
    hsd = _hnode_stage(msum, cnt, xpr, svsum, denom, Wl, bl, Wpack, e6)

    logits = (jnp.take(hsd[:, 0:3], src_c, axis=0)
              + jnp.take(hsd[:, 3:6], dst_c, axis=0) + le[:, :3])
    alpha_n = s[:, 0] * jnp.take(hsd[:, 6], dst_c, axis=0)
    return (logits, jax.lax.stop_gradient(alpha_n))


# speedup vs baseline: 3.0212x; 2.0905x over previous
import functools
import math

import jax
import jax.numpy as jnp
from jax import lax
from jax.experimental import pallas as pl
from jax.experimental.pallas import tpu as pltpu
from jax.experimental.pallas import tpu_sc as plsc

N_NODES = 50000
HD = 64
ED = 16
E_CITES = 800000
E_WRITES = 800000

_NODE_BLK = 2000
_EDGE_BLK = 6400

_NW = 32
_SC_CHUNK = 200


def _sc_mesh():
    return plsc.VectorSubcoreMesh(core_axis_name="c", subcore_axis_name="s")


def _qkv_gather_body(q_hbm, k_hbm, v_hbm, dst_hbm, src_hbm,
                     qk_hbm, vsrc_hbm,
                     dstv, srcv, qr, kr, vr, sem0, sem1, sem2):
    wid = lax.axis_index("s") * 2 + lax.axis_index("c")
    per_w = E_CITES // _NW
    nchunk = per_w // _SC_CHUNK
    C = _SC_CHUNK

    def chunk(ci, carry):
        base = wid * per_w + ci * C
        pltpu.sync_copy(dst_hbm.at[pl.ds(base, C)], dstv)
        pltpu.sync_copy(src_hbm.at[pl.ds(base, C)], srcv)
        cp0 = pltpu.async_copy(q_hbm.at[dstv], qr, sem0)
        cp1 = pltpu.async_copy(k_hbm.at[srcv], kr, sem1)
        cp2 = pltpu.async_copy(v_hbm.at[srcv], vr, sem2)
        cp0.wait()
        cp1.wait()

        def mul(r, carry2):
            for g in range(4):
                sl = pl.ds(g * 16, 16)
                qr[r, sl] = qr[r, sl] * kr[r, sl]
            return carry2

        lax.fori_loop(0, C, mul, 0)
        pltpu.sync_copy(qr, qk_hbm.at[pl.ds(base, C)])
        cp2.wait()
        pltpu.sync_copy(vr, vsrc_hbm.at[pl.ds(base, C)])
        return carry

    lax.fori_loop(0, nchunk, chunk, 0)


def _gather_rows_body(ntab, D, E, *refs):
    tab_hbm = refs[:ntab]
    idx_hbm = refs[ntab:2 * ntab]
    out_hbm = refs[2 * ntab:3 * ntab]
    idx_v = refs[3 * ntab:4 * ntab]
    row_v = refs[4 * ntab:5 * ntab]
    sems = refs[5 * ntab:6 * ntab]
    wid = lax.axis_index("s") * 2 + lax.axis_index("c")
    per_w = E // _NW
    C = _SC_CHUNK
    nchunk = per_w // C

    def chunk(ci, carry):
        base = wid * per_w + ci * C
        for t in range(ntab):
            pltpu.sync_copy(idx_hbm[t].at[pl.ds(base, C)], idx_v[t])
        cps = [pltpu.async_copy(tab_hbm[t].at[idx_v[t]], row_v[t], sems[t])
               for t in range(ntab)]
        for t in range(ntab):
            cps[t].wait()
            pltpu.sync_copy(row_v[t], out_hbm[t].at[pl.ds(base, C)])
        return carry

    lax.fori_loop(0, nchunk, chunk, 0)


def _gather_rows(tables, idxs):
    ntab = len(tables)
    E = idxs[0].shape[0]
    C = _SC_CHUNK
    f = pl.kernel(
        functools.partial(_gather_rows_body, ntab, None, E),
        mesh=_sc_mesh(),
        compiler_params=pltpu.CompilerParams(use_tc_tiling_on_sc=False),
        out_type=[jax.ShapeDtypeStruct((E, t.shape[1]), jnp.float32)
                  for t in tables],
        scratch_types=([pltpu.VMEM((C,), jnp.int32) for _ in tables]
                       + [pltpu.VMEM((C, t.shape[1]), jnp.float32)
                          for t in tables]
                       + [pltpu.SemaphoreType.DMA for _ in tables]),
    )
    return f(*tables, *idxs)


def _qkv_gather(q, k, v, dst_c, src_c):
    C = _SC_CHUNK
    f = pl.kernel(
        _qkv_gather_body,
        mesh=_sc_mesh(),
        compiler_params=pltpu.CompilerParams(use_tc_tiling_on_sc=False),
        out_type=[jax.ShapeDtypeStruct((E_CITES, HD), jnp.float32),
                  jax.ShapeDtypeStruct((E_CITES, HD), jnp.float32)],
        scratch_types=[
            pltpu.VMEM((C,), jnp.int32),
            pltpu.VMEM((C,), jnp.int32),
            pltpu.VMEM((C, HD), jnp.float32),
            pltpu.VMEM((C, HD), jnp.float32),
            pltpu.VMEM((C, HD), jnp.float32),
            pltpu.SemaphoreType.DMA,
            pltpu.SemaphoreType.DMA,
            pltpu.SemaphoreType.DMA,
        ],
    )
    return f(q, k, v, dst_c, src_c)


def _nodes_body(xpa_ref, xau_ref, Wp_ref, bp_ref, Wa_ref, ba_ref,
                Wq_ref, bq_ref, Wk_ref, bk_ref, Wv_ref, bv_ref, Wr_ref,
                xa_out, q_out, k_out, v_out, xpr_out):
    xp = jnp.maximum(xpa_ref[...] @ Wp_ref[...] + bp_ref[...][None, :], 0.0)
    xa = jnp.maximum(xau_ref[...] @ Wa_ref[...] + ba_ref[...][None, :], 0.0)
    xa_out[...] = xa
    q_out[...] = xp @ Wq_ref[...] + bq_ref[...][None, :]
    k_out[...] = xp @ Wk_ref[...] + bk_ref[...][None, :]
    v_out[...] = xp @ Wv_ref[...] + bv_ref[...][None, :]
    xpr_out[...] = xp @ Wr_ref[...]


def _nodes_stage(x_paper, x_author, Wp, bp, Wa, ba, Wq, bq, Wk, bk, Wv, bv, Wr):
    n = x_paper.shape[0]
    grid = (n // _NODE_BLK,)
    row_spec = pl.BlockSpec((_NODE_BLK, 128), lambda i: (i, 0))
    w128 = pl.BlockSpec((128, HD), lambda i: (0, 0))
    w64 = pl.BlockSpec((HD, HD), lambda i: (0, 0))
    b_spec = pl.BlockSpec((HD,), lambda i: (0,))
    out_spec = pl.BlockSpec((_NODE_BLK, HD), lambda i: (i, 0))
    out_shape = jax.ShapeDtypeStruct((n, HD), jnp.float32)
    return pl.pallas_call(
        _nodes_body,
        grid=grid,
        in_specs=[row_spec, row_spec, w128, b_spec, w128, b_spec,
                  w64, b_spec, w64, b_spec, w64, b_spec, w64],
        out_specs=[out_spec] * 5,
        out_shape=[out_shape] * 5,
    )(x_paper, x_author, Wp, bp, Wa, ba, Wq, bq, Wk, bk, Wv, bv, Wr)


def _edge_body(qk_ref, vsrc_ref, ea_ref, dt_ref, We_ref, be_ref,
               Wie_ref, wit_ref, bi4_ref, nl_ref,
               s_out, sv_out, le_out):
    ea = ea_ref[...]
    t = ea @ We_ref[...] + be_ref[...][None, :]
    a = jnp.sum(qk_ref[...] * t, axis=1, keepdims=True) * (1.0 / math.sqrt(HD))
    dt = dt_ref[...]
    dec = jnp.exp(nl_ref[0, 0] * dt)
    s = jnp.exp(a * dec)
    s_out[...] = s
    sv_out[...] = s * vsrc_ref[...]
    le_out[...] = ea @ Wie_ref[...] + dt * wit_ref[...] + bi4_ref[...]


def _edge_stage(qk, vsrc, ea, dt, We, be, Wie4, wit4, bi4, neg_lam):
    e = qk.shape[0]
    grid = (e // _EDGE_BLK,)
    rows64 = pl.BlockSpec((_EDGE_BLK, HD), lambda i: (i, 0))
    rows16 = pl.BlockSpec((_EDGE_BLK, ED), lambda i: (i, 0))
    vec = pl.BlockSpec((_EDGE_BLK, 1), lambda i: (i, 0))
    const = lambda shape: pl.BlockSpec(shape, lambda i: (0,) * len(shape))
    return pl.pallas_call(
        _edge_body,
        grid=grid,
        in_specs=[rows64, rows64, rows16, vec, const((ED, HD)), const((HD,)),
                  const((ED, 4)), const((1, 4)), const((1, 4)), const((1, 1))],
        out_specs=[vec, rows64, pl.BlockSpec((_EDGE_BLK, 4), lambda i: (i, 0))],
        out_shape=[jax.ShapeDtypeStruct((e, 1), jnp.float32),
                   jax.ShapeDtypeStruct((e, HD), jnp.float32),
                   jax.ShapeDtypeStruct((e, 4), jnp.float32)],
    )(qk, vsrc, ea, dt, We, be, Wie4, wit4, bi4, neg_lam)


def _hnode_body(msum_ref, cnt_ref, xpr_ref, svsum_ref, denom_ref,
                Wl_ref, bl_ref, Wpack_ref, e6_ref, hsd_out):
    mean = msum_ref[...] / jnp.maximum(cnt_ref[...], 1.0)
    sage = mean @ Wl_ref[...] + bl_ref[...][None, :] + xpr_ref[...]
    rden = 1.0 / jnp.maximum(denom_ref[...], 1e-30)
    h = jnp.maximum(sage + svsum_ref[...] * rden, 0.0)
    hsd_out[...] = h @ Wpack_ref[...] + rden * e6_ref[...]


def _hnode_stage(msum, cnt, xpr, svsum, denom, Wl, bl, Wpack, e6):
    n = msum.shape[0]
    grid = (n // _NODE_BLK,)
    rows64 = pl.BlockSpec((_NODE_BLK, HD), lambda i: (i, 0))
    vec = pl.BlockSpec((_NODE_BLK, 1), lambda i: (i, 0))
    const = lambda shape: pl.BlockSpec(shape, lambda i: (0,) * len(shape))
    return pl.pallas_call(
        _hnode_body,
        grid=grid,
        in_specs=[rows64, vec, rows64, rows64, vec,
                  const((HD, HD)), const((HD,)), const((HD, 16)), const((1, 16))],
        out_specs=pl.BlockSpec((_NODE_BLK, 16), lambda i: (i, 0)),
        out_shape=jax.ShapeDtypeStruct((n, 16), jnp.float32),
    )(msum, cnt, xpr, svsum, denom, Wl, bl, Wpack, e6)


def kernel(x_paper, x_author, edge_index_writes, edge_index_cites,
           edge_attr_cites, delta_t_cites, Wp, bp, Wa, ba, Wl, bl, Wr,
           Wq, bq, Wk, bk, Wv, bv, We, be, lam, Wi, bi):
    src_w = edge_index_writes[0]
    dst_w = edge_index_writes[1]
    src_c = edge_index_cites[0]
    dst_c = edge_index_cites[1]
    dt = delta_t_cites

    xa, q, k, v, xpr = _nodes_stage(
        x_paper, x_author, Wp, bp, Wa, ba, Wq, bq, Wk, bk, Wv, bv, Wr)

    (xg,) = _gather_rows([xa], [src_w])
    msum = jax.ops.segment_sum(xg, dst_w, num_segments=N_NODES)
    cnt = jax.ops.segment_sum(jnp.ones((E_WRITES,), jnp.float32), dst_w,
                              num_segments=N_NODES)[:, None]

    qk, vsrc = _qkv_gather(q, k, v, dst_c, src_c)

    Wie4 = jnp.zeros((ED, 4), jnp.float32).at[:, :3].set(Wi[2 * HD:2 * HD + ED, :])
    wit4 = jnp.zeros((1, 4), jnp.float32).at[0, :3].set(Wi[2 * HD + ED, :])
    bi4 = jnp.zeros((1, 4), jnp.float32).at[0, :3].set(bi)
    neg_lam = (-jnp.abs(lam)).reshape(1, 1)
    s, sv, le = _edge_stage(qk, vsrc, edge_attr_cites, dt, We, be,
                            Wie4, wit4, bi4, neg_lam)

    denom = jax.ops.segment_sum(s[:, 0], dst_c, num_segments=N_NODES)[:, None]
    svsum = jax.ops.segment_sum(sv, dst_c, num_segments=N_NODES)

    Wpack = (jnp.zeros((HD, 16), jnp.float32)
             .at[:, 0:3].set(Wi[:HD, :])
             .at[:, 3:6].set(Wi[HD:2 * HD, :]))
    e6 = jnp.zeros((1, 16), jnp.float32).at[0, 6].set(1.0)
    hsd = _hnode_stage(msum, cnt, xpr, svsum, denom, Wl, bl, Wpack, e6)

    hsrc, hdst = _gather_rows([hsd, hsd], [src_c, dst_c])
    logits = hsrc[:, 0:3] + hdst[:, 3:6] + le[:, :3]
    alpha_n = s[:, 0] * hdst[:, 6]
    return (logits, jax.lax.stop_gradient(alpha_n))


# speedup vs baseline: 4.3482x; 1.4392x over previous
import functools
import math

import jax
import jax.numpy as jnp
from jax import lax
from jax.experimental import pallas as pl
from jax.experimental.pallas import tpu as pltpu
from jax.experimental.pallas import tpu_sc as plsc

N_NODES = 50000
HD = 64
ED = 16
E_CITES = 800000
E_WRITES = 800000

_NODE_BLK = 2000
_EDGE_BLK = 6400

_NW = 32
_SC_CHUNK = 200


def _sc_mesh():
    return plsc.VectorSubcoreMesh(core_axis_name="c", subcore_axis_name="s")


def _qkv_gather_body(q_hbm, k_hbm, v_hbm, dst_hbm, src_hbm,
                     qk_hbm, vsrc_hbm,
                     dstv, srcv, qr, kr, vr, sem0, sem1, sem2):
    wid = lax.axis_index("s") * 2 + lax.axis_index("c")
    per_w = E_CITES // _NW
    nchunk = per_w // _SC_CHUNK
    C = _SC_CHUNK

    def chunk(ci, carry):
        base = wid * per_w + ci * C
        pltpu.sync_copy(dst_hbm.at[pl.ds(base, C)], dstv)
        pltpu.sync_copy(src_hbm.at[pl.ds(base, C)], srcv)
        cp0 = pltpu.async_copy(q_hbm.at[dstv], qr, sem0)
        cp1 = pltpu.async_copy(k_hbm.at[srcv], kr, sem1)
        cp2 = pltpu.async_copy(v_hbm.at[srcv], vr, sem2)
        cp0.wait()
        cp1.wait()

        def mul(r, carry2):
            for g in range(4):
                sl = pl.ds(g * 16, 16)
                qr[r, sl] = qr[r, sl] * kr[r, sl]
            return carry2

        lax.fori_loop(0, C, mul, 0)
        pltpu.sync_copy(qr, qk_hbm.at[pl.ds(base, C)])
        cp2.wait()
        pltpu.sync_copy(vr, vsrc_hbm.at[pl.ds(base, C)])
        return carry

    lax.fori_loop(0, nchunk, chunk, 0)


def _gather_rows_body(ntab, D, E, *refs):
    tab_hbm = refs[:ntab]
    idx_hbm = refs[ntab:2 * ntab]
    out_hbm = refs[2 * ntab:3 * ntab]
    idx_v = refs[3 * ntab:4 * ntab]
    row_v = refs[4 * ntab:5 * ntab]
    sems = refs[5 * ntab:6 * ntab]
    wid = lax.axis_index("s") * 2 + lax.axis_index("c")
    per_w = E // _NW
    C = _SC_CHUNK
    nchunk = per_w // C

    def chunk(ci, carry):
        base = wid * per_w + ci * C
        for t in range(ntab):
            pltpu.sync_copy(idx_hbm[t].at[pl.ds(base, C)], idx_v[t])
        cps = [pltpu.async_copy(tab_hbm[t].at[idx_v[t]], row_v[t], sems[t])
               for t in range(ntab)]
        for t in range(ntab):
            cps[t].wait()
            pltpu.sync_copy(row_v[t], out_hbm[t].at[pl.ds(base, C)])
        return carry

    lax.fori_loop(0, nchunk, chunk, 0)


def _gather_rows(tables, idxs):
    ntab = len(tables)
    E = idxs[0].shape[0]
    C = _SC_CHUNK
    f = pl.kernel(
        functools.partial(_gather_rows_body, ntab, None, E),
        mesh=_sc_mesh(),
        compiler_params=pltpu.CompilerParams(use_tc_tiling_on_sc=False),
        out_type=[jax.ShapeDtypeStruct((E, t.shape[1]), jnp.float32)
                  for t in tables],
        scratch_types=([pltpu.VMEM((C,), jnp.int32) for _ in tables]
                       + [pltpu.VMEM((C, t.shape[1]), jnp.float32)
                          for t in tables]
                       + [pltpu.SemaphoreType.DMA for _ in tables]),
    )
    return f(*tables, *idxs)


_HALF = N_NODES // 2
_QTR = N_NODES // 4
_ACC_ROWS = 12512
_TILE_ROWS = _ACC_ROWS // 16
_ROWW = HD + 16
_SCAT_CHUNK = 400


def _seg_scatter_body(gather_rows, E, rows_hbm, w_hbm, dst_hbm,
                      zeros80, sum_out,
                      idxv, adjv, rowr, wv, acc, semg):
    cid = lax.axis_index("c")
    sid = lax.axis_index("s")
    C = _SCAT_CHUNK
    per_tile = E // 16
    nchunk = per_tile // C

    for p in range(2):
        pltpu.sync_copy(zeros80, acc.at[pl.ds(sid * _TILE_ROWS, _TILE_ROWS)])
        plsc.subcore_barrier()
        lo = cid * _HALF + p * _QTR

        def rchunk(ci, carry):
            base = sid * per_tile + ci * C
            pltpu.sync_copy(dst_hbm.at[pl.ds(base, C)], idxv)
            if gather_rows:
                pltpu.sync_copy(w_hbm.at[pl.ds(base, C)], wv)
                cp = pltpu.async_copy(rows_hbm.at[wv], rowr, semg)
            else:
                cp = pltpu.async_copy(rows_hbm.at[pl.ds(base, C)], rowr, semg)

            def lanes(j, carry2):
                sl = pl.ds(j * 16, 16)
                adj = idxv[sl] - lo
                m = (adj >= 0) & (adj < _QTR)
                adjv[sl] = jnp.where(m, adj, _QTR + 2)
                return carry2

            lax.fori_loop(0, C // 16, lanes, 0)
            cp.wait()
            pltpu.sync_copy(rowr, acc.at[adjv], add=True)
            return carry

        lax.fori_loop(0, nchunk, rchunk, 0)
        plsc.subcore_barrier()
        pltpu.sync_copy(acc.at[pl.ds(sid * _TILE_ROWS, _TILE_ROWS)],
                        sum_out.at[cid, p, pl.ds(sid * _TILE_ROWS, _TILE_ROWS)])


def _seg_scatter(rows80, idx, dst, gather_rows):
    E = dst.shape[0]
    C = _SCAT_CHUNK
    f = pl.kernel(
        functools.partial(_seg_scatter_body, gather_rows, E),
        mesh=_sc_mesh(),
        compiler_params=pltpu.CompilerParams(use_tc_tiling_on_sc=False),
        out_type=jax.ShapeDtypeStruct((2, 2, _ACC_ROWS, _ROWW), jnp.float32),
        scratch_types=[
            pltpu.VMEM((C,), jnp.int32),
            pltpu.VMEM((C,), jnp.int32),
            pltpu.VMEM((C, _ROWW), jnp.float32),
            pltpu.VMEM((C,), jnp.int32),
            pltpu.VMEM_SHARED((_ACC_ROWS, _ROWW), jnp.float32),
            pltpu.SemaphoreType.DMA,
        ],
    )
    zeros80 = jnp.zeros((_TILE_ROWS, _ROWW), jnp.float32)
    sum_p = f(rows80, idx, dst, zeros80)
    seg80 = jnp.concatenate([sum_p[0, 0, :_QTR], sum_p[0, 1, :_QTR],
                             sum_p[1, 0, :_QTR], sum_p[1, 1, :_QTR]], axis=0)
    return seg80[:, :HD], seg80[:, HD]


def _qkv_gather(q, k, v, dst_c, src_c):
    C = _SC_CHUNK
    f = pl.kernel(
        _qkv_gather_body,
        mesh=_sc_mesh(),
        compiler_params=pltpu.CompilerParams(use_tc_tiling_on_sc=False),
        out_type=[jax.ShapeDtypeStruct((E_CITES, HD), jnp.float32),
                  jax.ShapeDtypeStruct((E_CITES, HD), jnp.float32)],
        scratch_types=[
            pltpu.VMEM((C,), jnp.int32),
            pltpu.VMEM((C,), jnp.int32),
            pltpu.VMEM((C, HD), jnp.float32),
            pltpu.VMEM((C, HD), jnp.float32),
            pltpu.VMEM((C, HD), jnp.float32),
            pltpu.SemaphoreType.DMA,
            pltpu.SemaphoreType.DMA,
            pltpu.SemaphoreType.DMA,
        ],
    )
    return f(q, k, v, dst_c, src_c)


def _nodes_body(xpa_ref, xau_ref, Wp_ref, bp_ref, Wa_ref, ba_ref,
                Wq_ref, bq_ref, Wk_ref, bk_ref, Wv_ref, bv_ref, Wr_ref,
                xa_out, q_out, k_out, v_out, xpr_out):
    xp = jnp.maximum(xpa_ref[...] @ Wp_ref[...] + bp_ref[...][None, :], 0.0)
    xa = jnp.maximum(xau_ref[...] @ Wa_ref[...] + ba_ref[...][None, :], 0.0)
    n = xa.shape[0]
    xa_out[...] = jnp.concatenate(
        [xa, jnp.ones((n, 1), jnp.float32), jnp.zeros((n, 15), jnp.float32)],
        axis=1)
    q_out[...] = xp @ Wq_ref[...] + bq_ref[...][None, :]
    k_out[...] = xp @ Wk_ref[...] + bk_ref[...][None, :]
    v_out[...] = xp @ Wv_ref[...] + bv_ref[...][None, :]
    xpr_out[...] = xp @ Wr_ref[...]


def _nodes_stage(x_paper, x_author, Wp, bp, Wa, ba, Wq, bq, Wk, bk, Wv, bv, Wr):
    n = x_paper.shape[0]
    grid = (n // _NODE_BLK,)
    row_spec = pl.BlockSpec((_NODE_BLK, 128), lambda i: (i, 0))
    w128 = pl.BlockSpec((128, HD), lambda i: (0, 0))
    w64 = pl.BlockSpec((HD, HD), lambda i: (0, 0))
    b_spec = pl.BlockSpec((HD,), lambda i: (0,))
    out_spec = pl.BlockSpec((_NODE_BLK, HD), lambda i: (i, 0))
    out80_spec = pl.BlockSpec((_NODE_BLK, _ROWW), lambda i: (i, 0))
    out_shape = jax.ShapeDtypeStruct((n, HD), jnp.float32)
    out80_shape = jax.ShapeDtypeStruct((n, _ROWW), jnp.float32)
    return pl.pallas_call(
        _nodes_body,
        grid=grid,
        in_specs=[row_spec, row_spec, w128, b_spec, w128, b_spec,
                  w64, b_spec, w64, b_spec, w64, b_spec, w64],
        out_specs=[out80_spec] + [out_spec] * 4,
        out_shape=[out80_shape] + [out_shape] * 4,
    )(x_paper, x_author, Wp, bp, Wa, ba, Wq, bq, Wk, bk, Wv, bv, Wr)


def _edge_body(qk_ref, vsrc_ref, ea_ref, dt_ref, We_ref, be_ref,
               Wie_ref, wit_ref, bi4_ref, nl_ref,
               s_out, sv_out, le_out):
    ea = ea_ref[...]
    t = ea @ We_ref[...] + be_ref[...][None, :]
    a = jnp.sum(qk_ref[...] * t, axis=1, keepdims=True) * (1.0 / math.sqrt(HD))
    dt = dt_ref[...]
    dec = jnp.exp(nl_ref[0, 0] * dt)
    s = jnp.exp(a * dec)
    s_out[...] = s
    n = s.shape[0]
    sv_out[...] = jnp.concatenate(
        [s * vsrc_ref[...], s, jnp.zeros((n, 15), jnp.float32)], axis=1)
    le_out[...] = ea @ Wie_ref[...] + dt * wit_ref[...] + bi4_ref[...]


def _edge_stage(qk, vsrc, ea, dt, We, be, Wie4, wit4, bi4, neg_lam):
    e = qk.shape[0]
    grid = (e // _EDGE_BLK,)
    rows64 = pl.BlockSpec((_EDGE_BLK, HD), lambda i: (i, 0))
    rows16 = pl.BlockSpec((_EDGE_BLK, ED), lambda i: (i, 0))
    vec = pl.BlockSpec((_EDGE_BLK, 1), lambda i: (i, 0))
    const = lambda shape: pl.BlockSpec(shape, lambda i: (0,) * len(shape))
    return pl.pallas_call(
        _edge_body,
        grid=grid,
        in_specs=[rows64, rows64, rows16, vec, const((ED, HD)), const((HD,)),
                  const((ED, 4)), const((1, 4)), const((1, 4)), const((1, 1))],
        out_specs=[vec, pl.BlockSpec((_EDGE_BLK, _ROWW), lambda i: (i, 0)),
                   pl.BlockSpec((_EDGE_BLK, 4), lambda i: (i, 0))],
        out_shape=[jax.ShapeDtypeStruct((e, 1), jnp.float32),
                   jax.ShapeDtypeStruct((e, _ROWW), jnp.float32),
                   jax.ShapeDtypeStruct((e, 4), jnp.float32)],
    )(qk, vsrc, ea, dt, We, be, Wie4, wit4, bi4, neg_lam)


def _hnode_body(msum_ref, cnt_ref, xpr_ref, svsum_ref, denom_ref,
                Wl_ref, bl_ref, Wpack_ref, e6_ref, hsd_out):
    mean = msum_ref[...] / jnp.maximum(cnt_ref[...], 1.0)
    sage = mean @ Wl_ref[...] + bl_ref[...][None, :] + xpr_ref[...]
    rden = 1.0 / jnp.maximum(denom_ref[...], 1e-30)
    h = jnp.maximum(sage + svsum_ref[...] * rden, 0.0)
    hsd_out[...] = h @ Wpack_ref[...] + rden * e6_ref[...]


def _hnode_stage(msum, cnt, xpr, svsum, denom, Wl, bl, Wpack, e6):
    n = msum.shape[0]
    grid = (n // _NODE_BLK,)
    rows64 = pl.BlockSpec((_NODE_BLK, HD), lambda i: (i, 0))
    vec = pl.BlockSpec((_NODE_BLK, 1), lambda i: (i, 0))
    const = lambda shape: pl.BlockSpec(shape, lambda i: (0,) * len(shape))
    return pl.pallas_call(
        _hnode_body,
        grid=grid,
        in_specs=[rows64, vec, rows64, rows64, vec,
                  const((HD, HD)), const((HD,)), const((HD, 16)), const((1, 16))],
        out_specs=pl.BlockSpec((_NODE_BLK, 16), lambda i: (i, 0)),
        out_shape=jax.ShapeDtypeStruct((n, 16), jnp.float32),
    )(msum, cnt, xpr, svsum, denom, Wl, bl, Wpack, e6)


def kernel(x_paper, x_author, edge_index_writes, edge_index_cites,
           edge_attr_cites, delta_t_cites, Wp, bp, Wa, ba, Wl, bl, Wr,
           Wq, bq, Wk, bk, Wv, bv, We, be, lam, Wi, bi):
    src_w = edge_index_writes[0]
    dst_w = edge_index_writes[1]
    src_c = edge_index_cites[0]
    dst_c = edge_index_cites[1]
    dt = delta_t_cites

    xa80, q, k, v, xpr = _nodes_stage(
        x_paper, x_author, Wp, bp, Wa, ba, Wq, bq, Wk, bk, Wv, bv, Wr)

    msum, cnt = _seg_scatter(xa80, src_w, dst_w, gather_rows=True)
    cnt = cnt[:, None]

    qk, vsrc = _qkv_gather(q, k, v, dst_c, src_c)

    Wie4 = jnp.zeros((ED, 4), jnp.float32).at[:, :3].set(Wi[2 * HD:2 * HD + ED, :])
    wit4 = jnp.zeros((1, 4), jnp.float32).at[0, :3].set(Wi[2 * HD + ED, :])
    bi4 = jnp.zeros((1, 4), jnp.float32).at[0, :3].set(bi)
    neg_lam = (-jnp.abs(lam)).reshape(1, 1)
    s, sv80, le = _edge_stage(qk, vsrc, edge_attr_cites, dt, We, be,
                              Wie4, wit4, bi4, neg_lam)

    svsum, denom = _seg_scatter(sv80, src_c, dst_c, gather_rows=False)
    denom = denom[:, None]

    Wpack = (jnp.zeros((HD, 16), jnp.float32)
             .at[:, 0:3].set(Wi[:HD, :])
             .at[:, 3:6].set(Wi[HD:2 * HD, :]))
    e6 = jnp.zeros((1, 16), jnp.float32).at[0, 6].set(1.0)
    hsd = _hnode_stage(msum, cnt, xpr, svsum, denom, Wl, bl, Wpack, e6)

    hsrc, hdst = _gather_rows([hsd, hsd], [src_c, dst_c])
    logits = hsrc[:, 0:3] + hdst[:, 3:6] + le[:, :3]
    alpha_n = s[:, 0] * hdst[:, 6]
    return (logits, jax.lax.stop_gradient(alpha_n))
